# baseline (speedup 1.0000x reference)
import functools

import jax
import jax.numpy as jnp
from jax import lax
from jax.experimental import pallas as pl
from jax.experimental.pallas import tpu as pltpu
from jax.experimental.pallas import tpu_sc as plsc

VOCAB = 1000000
DIM = 64
BATCH = 16384
NEG = 20

NC = 2
NS = 16
NW = NC * NS

BW = BATCH // NW
NEGW = BATCH * NEG // NW
CHUNK = 128


def _sc_gather_body(cw_hbm, xw_hbm, nw_hbm, ie_hbm, oe_hbm,
                    outc_hbm, outx_hbm, outn_hbm,
                    idx_c, idx_x, idx_n, buf0, buf1, sem0, sem1):
    c = lax.axis_index("c")
    s = lax.axis_index("s")
    wid = s * NC + c
    cbase = wid * BW
    nbase = wid * NEGW

    pltpu.sync_copy(cw_hbm.at[pl.ds(cbase, BW)], idx_c)
    pltpu.sync_copy(xw_hbm.at[pl.ds(cbase, BW)], idx_x)
    pltpu.sync_copy(nw_hbm.at[pl.ds(nbase, NEGW)], idx_n)

    def pair(table, idx_ref, out_hbm, out_base, j):
        o0 = j * (2 * CHUNK)
        cp0 = pltpu.async_copy(
            table.at[idx_ref.at[pl.ds(o0, CHUNK)]], buf0, sem0)
        cp1 = pltpu.async_copy(
            table.at[idx_ref.at[pl.ds(o0 + CHUNK, CHUNK)]], buf1, sem1)
        cp0.wait()
        pltpu.sync_copy(buf0, out_hbm.at[pl.ds(out_base + o0, CHUNK)])
        cp1.wait()
        pltpu.sync_copy(buf1, out_hbm.at[pl.ds(out_base + o0 + CHUNK, CHUNK)])

    for j in range(BW // (2 * CHUNK)):
        pair(ie_hbm, idx_c, outc_hbm, cbase, j)
    for j in range(BW // (2 * CHUNK)):
        pair(oe_hbm, idx_x, outx_hbm, cbase, j)

    def nbody(j, carry):
        pair(oe_hbm, idx_n, outn_hbm, nbase, j)
        return carry

    lax.fori_loop(0, NEGW // (2 * CHUNK), nbody, 0)


@jax.jit
def kernel(center_words, context_words, negative_words,
           input_embeddings, output_embeddings):
    neg_flat = negative_words.reshape(BATCH * NEG)
    mesh = plsc.VectorSubcoreMesh(core_axis_name="c", subcore_axis_name="s")
    run = pl.kernel(
        _sc_gather_body,
        mesh=mesh,
        compiler_params=pltpu.CompilerParams(use_tc_tiling_on_sc=False),
        out_type=[
            jax.ShapeDtypeStruct((BATCH, DIM), jnp.float32),
            jax.ShapeDtypeStruct((BATCH, DIM), jnp.float32),
            jax.ShapeDtypeStruct((BATCH * NEG, DIM), jnp.float32),
        ],
        scratch_types=[
            pltpu.VMEM((BW,), jnp.int32),
            pltpu.VMEM((BW,), jnp.int32),
            pltpu.VMEM((NEGW,), jnp.int32),
            pltpu.VMEM((CHUNK, DIM), jnp.float32),
            pltpu.VMEM((CHUNK, DIM), jnp.float32),
            pltpu.SemaphoreType.DMA,
            pltpu.SemaphoreType.DMA,
        ],
    )
    center, context, negatives = run(
        center_words.astype(jnp.int32),
        context_words.astype(jnp.int32),
        neg_flat.astype(jnp.int32),
        input_embeddings,
        output_embeddings,
    )
    return center, context, negatives.reshape(BATCH, NEG, DIM)


# speedup vs baseline: 1.0140x; 1.0140x over previous
import functools

import jax
import jax.numpy as jnp
from jax import lax
from jax.experimental import pallas as pl
from jax.experimental.pallas import tpu as pltpu
from jax.experimental.pallas import tpu_sc as plsc

VOCAB = 1000000
DIM = 64
BATCH = 16384
NEG = 20

NC = 2
NS = 16
NW = NC * NS

BW = BATCH // NW
NEGW = BATCH * NEG // NW
CHUNK = 512


def _sc_gather_body(cw_hbm, xw_hbm, nw_hbm, ie_hbm, oe_hbm,
                    outc_hbm, outx_hbm, outn_hbm,
                    idx_c, idx_x, idx_n, buf0, buf1, sem0, sem1):
    c = lax.axis_index("c")
    s = lax.axis_index("s")
    wid = s * NC + c
    cbase = wid * BW
    nbase = wid * NEGW

    pltpu.sync_copy(cw_hbm.at[pl.ds(cbase, BW)], idx_c)
    pltpu.sync_copy(xw_hbm.at[pl.ds(cbase, BW)], idx_x)
    pltpu.sync_copy(nw_hbm.at[pl.ds(nbase, NEGW)], idx_n)

    def pair(table, idx_ref, out_hbm, out_base, j):
        o0 = j * (2 * CHUNK)
        cp0 = pltpu.async_copy(
            table.at[idx_ref.at[pl.ds(o0, CHUNK)]], buf0, sem0)
        cp1 = pltpu.async_copy(
            table.at[idx_ref.at[pl.ds(o0 + CHUNK, CHUNK)]], buf1, sem1)
        cp0.wait()
        pltpu.sync_copy(buf0, out_hbm.at[pl.ds(out_base + o0, CHUNK)])
        cp1.wait()
        pltpu.sync_copy(buf1, out_hbm.at[pl.ds(out_base + o0 + CHUNK, CHUNK)])

    cpc = pltpu.async_copy(ie_hbm.at[idx_c], buf0, sem0)
    cpx = pltpu.async_copy(oe_hbm.at[idx_x], buf1, sem1)
    cpc.wait()
    pltpu.sync_copy(buf0, outc_hbm.at[pl.ds(cbase, BW)])
    cpx.wait()
    pltpu.sync_copy(buf1, outx_hbm.at[pl.ds(cbase, BW)])

    def nbody(j, carry):
        pair(oe_hbm, idx_n, outn_hbm, nbase, j)
        return carry

    lax.fori_loop(0, NEGW // (2 * CHUNK), nbody, 0)


@jax.jit
def kernel(center_words, context_words, negative_words,
           input_embeddings, output_embeddings):
    neg_flat = negative_words.reshape(BATCH * NEG)
    mesh = plsc.VectorSubcoreMesh(core_axis_name="c", subcore_axis_name="s")
    run = pl.kernel(
        _sc_gather_body,
        mesh=mesh,
        compiler_params=pltpu.CompilerParams(use_tc_tiling_on_sc=False),
        out_type=[
            jax.ShapeDtypeStruct((BATCH, DIM), jnp.float32),
            jax.ShapeDtypeStruct((BATCH, DIM), jnp.float32),
            jax.ShapeDtypeStruct((BATCH * NEG, DIM), jnp.float32),
        ],
        scratch_types=[
            pltpu.VMEM((BW,), jnp.int32),
            pltpu.VMEM((BW,), jnp.int32),
            pltpu.VMEM((NEGW,), jnp.int32),
            pltpu.VMEM((max(CHUNK, BW), DIM), jnp.float32),
            pltpu.VMEM((max(CHUNK, BW), DIM), jnp.float32),
            pltpu.SemaphoreType.DMA,
            pltpu.SemaphoreType.DMA,
        ],
    )
    center, context, negatives = run(
        center_words.astype(jnp.int32),
        context_words.astype(jnp.int32),
        neg_flat.astype(jnp.int32),
        input_embeddings,
        output_embeddings,
    )
    return center, context, negatives.reshape(BATCH, NEG, DIM)
